# baseline (speedup 1.0000x reference)
import jax
import jax.numpy as jnp
from jax import lax
from jax.experimental import pallas as pl
from jax.experimental.pallas import tpu as pltpu
from jax.experimental.pallas import tpu_sc as plsc

_D = 1024
_B = 8192 * 4
_NC, _NS = 2, 16
_NW = _NC * _NS
_BPW = _B // _NW
_G8 = 8
_NGRP = _BPW // _G8
_NROW = 263


def _body(table_hbm, out_hbm, shared_v, osem):
    wid = lax.axis_index("s") * _NC + lax.axis_index("c")
    base = wid * _BPW * _D

    out_cps = []
    for g in range(_NGRP):
        pos = (g * 7) % (_NROW - _G8)
        out_cps.append(
            pltpu.async_copy(
                shared_v.at[pl.ds(pos * _D, _G8 * _D)],
                out_hbm.at[pl.ds(base + g * _G8 * _D, _G8 * _D)],
                osem,
            )
        )
    for cp in out_cps:
        cp.wait()


@jax.jit
def _lookup(table):
    run = pl.kernel(
        _body,
        out_type=jax.ShapeDtypeStruct((_B * _D,), jnp.float32),
        mesh=plsc.VectorSubcoreMesh(core_axis_name="c", subcore_axis_name="s"),
        scratch_types=[
            pltpu.VMEM_SHARED((_NROW * _D,), jnp.float32),
            pltpu.SemaphoreType.DMA,
        ],
    )
    return run(table)


def kernel(seq_input, token_type_input, table):
    S, N = token_type_input.shape
    out = _lookup(table.reshape(-1))
    return out.reshape(S, N, _D)


# speedup vs baseline: 1.1174x; 1.1174x over previous
import jax
import jax.numpy as jnp
from jax import lax
from jax.experimental import pallas as pl
from jax.experimental.pallas import tpu as pltpu
from jax.experimental.pallas import tpu_sc as plsc

_TYPE_TOKEN_NUM = 2
_D = 1024
_B = 8192 * 4
_NC, _NS = 2, 16
_NW = _NC * _NS
_BPW = _B // _NW
_G = 4
_NGRP = _BPW // _G
_NCOMBO = _TYPE_TOKEN_NUM ** _G

_DB_SEQ = (0, 0, 0, 0, 1, 1, 1, 1, 0, 1, 1, 0, 0, 1, 0, 1, 0, 0, 0)
_POS_LUT = (0, 1, 11, 2, 14, 12, 8, 3, 15, 10, 13, 7, 9, 6, 5, 4)
_NROW = len(_DB_SEQ)


def _body(table_hbm, idx_hbm, out_hbm, idx_v, combo_v, kid_v, csem, osem):
    wid = lax.axis_index("s") * _NC + lax.axis_index("c")
    base = wid * _BPW * _D

    combo_cps = []
    for i, bit in enumerate(_DB_SEQ):
        combo_cps.append(
            pltpu.async_copy(
                table_hbm.at[wid, pl.ds(bit * _D, _D)],
                combo_v.at[pl.ds(i * _D, _D)],
                csem,
            )
        )

    pltpu.sync_copy(idx_hbm.at[wid], idx_v)

    for j in range(_NGRP // 16):
        k_vec = idx_v[0, pl.ds(16 * j, 16)]
        for p in range(1, _G):
            k_vec = k_vec * 2 + idx_v[p, pl.ds(16 * j, 16)]
        pos = jnp.zeros((16,), jnp.int32)
        for k in range(_NCOMBO):
            if _POS_LUT[k]:
                pos = pos + jnp.where(
                    k_vec == k, jnp.int32(_POS_LUT[k] * _D), jnp.int32(0)
                )
        kid_v[pl.ds(16 * j, 16)] = pos

    for cp in combo_cps:
        cp.wait()

    out_cps = []
    for g in range(_NGRP):
        if g % 16 == 0:
            k_vec = kid_v[pl.ds(g, 16)]
        k = pl.multiple_of(k_vec[g % 16], _D)
        out_cps.append(
            pltpu.async_copy(
                combo_v.at[pl.ds(k, _G * _D)],
                out_hbm.at[pl.ds(base + g * _G * _D, _G * _D)],
                osem,
            )
        )
    for cp in out_cps:
        cp.wait()


@jax.jit
def _lookup(table, idx3):
    run = pl.kernel(
        _body,
        out_type=jax.ShapeDtypeStruct((_B * _D,), jnp.float32),
        mesh=plsc.VectorSubcoreMesh(core_axis_name="c", subcore_axis_name="s"),
        scratch_types=[
            pltpu.VMEM((_G, _NGRP), jnp.int32),
            pltpu.VMEM((_NROW * _D,), jnp.float32),
            pltpu.VMEM((_NGRP,), jnp.int32),
            pltpu.SemaphoreType.DMA,
            pltpu.SemaphoreType.DMA,
        ],
    )
    return run(table, idx3)


def kernel(seq_input, token_type_input, table):
    S, N = token_type_input.shape
    idx3 = token_type_input.reshape(_NW, _NGRP, _G).transpose(0, 2, 1)
    table_rep = jnp.tile(table.reshape(1, -1), (_NW, 1))
    out = _lookup(table_rep, idx3)
    return out.reshape(S, N, _D)
